# Initial kernel scaffold; baseline (speedup 1.0000x reference)
import functools

import jax
import jax.numpy as jnp
from jax import lax
from jax.experimental import pallas as pl
from jax.experimental.pallas import tpu as pltpu
from jax.experimental.pallas import tpu_sc as plsc

VOCAB = 64001
DIM = 128
MAX_LEN = 200
B = 4096
TOK = B * MAX_LEN
EPS = 1e-12


def _transform_body(w_ref, w2_ref, o_ref):
    o_ref[...] = lax.dot_general(
        w_ref[...], w2_ref[...], (((1,), (1,)), ((), ())),
        preferred_element_type=jnp.float32)


def _transform_table(word_table, W2):
    R = 512
    return pl.pallas_call(
        _transform_body,
        grid=(pl.cdiv(VOCAB, R),),
        in_specs=[pl.BlockSpec((R, DIM), lambda i: (i, 0)),
                  pl.BlockSpec((DIM, DIM), lambda i: (0, 0))],
        out_specs=pl.BlockSpec((R, DIM), lambda i: (i, 0)),
        out_shape=jax.ShapeDtypeStruct((VOCAB, DIM), jnp.float32),
    )(word_table, W2)


_NW = 32
_B_PER_W = TOK // _NW
_CH = 256
_NCH = _B_PER_W // _CH


def _sc_gather(table, ids2d):
    mesh = plsc.VectorSubcoreMesh(core_axis_name="c", subcore_axis_name="s")

    @functools.partial(
        pl.kernel,
        out_type=jax.ShapeDtypeStruct((TOK, DIM), jnp.float32),
        mesh=mesh,
        scratch_types=[
            pltpu.VMEM((2, 128), jnp.int32),
            pltpu.VMEM((_CH, DIM), jnp.float32),
            pltpu.SemaphoreType.DMA,
        ],
    )
    def k(t_hbm, ids_hbm, out_hbm, idx_v, rows_v, sem):
        wid = lax.axis_index("s") * 2 + lax.axis_index("c")

        def body(g, carry):
            base = wid * _B_PER_W + g * _CH
            irow = wid * (_B_PER_W // 128) + g * 2
            pltpu.sync_copy(ids_hbm.at[pl.ds(irow, 2)], idx_v)
            c0 = pltpu.async_copy(t_hbm.at[idx_v.at[0]],
                                  rows_v.at[pl.ds(0, 128)], sem)
            c1 = pltpu.async_copy(t_hbm.at[idx_v.at[1]],
                                  rows_v.at[pl.ds(128, 128)], sem)
            c0.wait()
            c1.wait()
            pltpu.sync_copy(rows_v, out_hbm.at[pl.ds(base, _CH)])
            return carry

        lax.fori_loop(0, _NCH, body, 0)

    return k(table, ids2d)


def _ln_body(g_ref, seg_ref, pos_ref, typ_ref, gam_ref, bet_ref, o_ref):
    x = g_ref[...] + pos_ref[...][None, :, :]
    seg = seg_ref[...]
    t = typ_ref[...]
    typ = jnp.where((seg == 1)[..., None], t[1],
                    jnp.where((seg == 2)[..., None], t[2], t[0]))
    x = x + typ
    mean = jnp.mean(x, axis=-1, keepdims=True)
    var = jnp.mean(jnp.square(x - mean), axis=-1, keepdims=True)
    y = (x - mean) * lax.rsqrt(var + EPS)
    o_ref[...] = y * gam_ref[...][0][None, None, :] + bet_ref[...][0][None, None, :]


def _ln(gathered, segment_ids, pos_table, type_table, gamma, beta):
    BR = 16
    return pl.pallas_call(
        _ln_body,
        grid=(B // BR,),
        in_specs=[
            pl.BlockSpec((BR, MAX_LEN, DIM), lambda i: (i, 0, 0)),
            pl.BlockSpec((BR, MAX_LEN), lambda i: (i, 0)),
            pl.BlockSpec((MAX_LEN, DIM), lambda i: (0, 0)),
            pl.BlockSpec((3, DIM), lambda i: (0, 0)),
            pl.BlockSpec((1, DIM), lambda i: (0, 0)),
            pl.BlockSpec((1, DIM), lambda i: (0, 0)),
        ],
        out_specs=pl.BlockSpec((BR, MAX_LEN, DIM), lambda i: (i, 0, 0)),
        out_shape=jax.ShapeDtypeStruct((B, MAX_LEN, DIM), jnp.float32),
    )(gathered, segment_ids, pos_table, type_table,
      gamma.reshape(1, DIM), beta.reshape(1, DIM))


def kernel(input_ids, segment_ids, word_table, W2, pos_table, type_table,
           gamma, beta):
    table = _transform_table(word_table, W2)
    ids2d = input_ids.astype(jnp.int32).reshape(TOK // 128, 128)
    gathered = _sc_gather(table, ids2d)
    return _ln(gathered.reshape(B, MAX_LEN, DIM), segment_ids.astype(jnp.int32),
               pos_table, type_table, gamma, beta)


# speedup vs baseline: 8.4772x; 8.4772x over previous
import functools

import jax
import jax.numpy as jnp
from jax import lax
from jax.experimental import pallas as pl
from jax.experimental.pallas import tpu as pltpu
from jax.experimental.pallas import tpu_sc as plsc

VOCAB = 64001
DIM = 128
MAX_LEN = 200
B = 4096
TOK = B * MAX_LEN
EPS = 1e-12


def _transform_body(w_ref, w2_ref, o_ref):
    o_ref[...] = lax.dot_general(
        w_ref[...], w2_ref[...], (((1,), (1,)), ((), ())),
        preferred_element_type=jnp.float32)


def _transform_table(word_table, W2):
    R = 512
    return pl.pallas_call(
        _transform_body,
        grid=(pl.cdiv(VOCAB, R),),
        in_specs=[pl.BlockSpec((R, DIM), lambda i: (i, 0)),
                  pl.BlockSpec((DIM, DIM), lambda i: (0, 0))],
        out_specs=pl.BlockSpec((R, DIM), lambda i: (i, 0)),
        out_shape=jax.ShapeDtypeStruct((VOCAB, DIM), jnp.float32),
    )(word_table, W2)


_NW = 32
_B_PER_W = TOK // _NW
_CH = 256
_NCH = _B_PER_W // _CH


def _sc_gather(table, ids2d):
    mesh = plsc.VectorSubcoreMesh(core_axis_name="c", subcore_axis_name="s")

    @functools.partial(
        pl.kernel,
        out_type=jax.ShapeDtypeStruct((TOK, DIM), jnp.float32),
        mesh=mesh,
        scratch_types=[
            pltpu.VMEM((2, 128), jnp.int32),
            pltpu.VMEM((_CH, DIM), jnp.float32),
            pltpu.SemaphoreType.DMA,
        ],
    )
    def k(t_hbm, ids_hbm, out_hbm, idx_v, rows_v, sem):
        wid = lax.axis_index("s") * 2 + lax.axis_index("c")

        def body(g, carry):
            base = wid * _B_PER_W + g * _CH
            irow = wid * (_B_PER_W // 128) + g * 2
            pltpu.sync_copy(ids_hbm.at[pl.ds(irow, 2)], idx_v)
            c0 = pltpu.async_copy(t_hbm.at[idx_v.at[0]],
                                  rows_v.at[pl.ds(0, 128)], sem)
            c1 = pltpu.async_copy(t_hbm.at[idx_v.at[1]],
                                  rows_v.at[pl.ds(128, 128)], sem)
            c0.wait()
            c1.wait()
            pltpu.sync_copy(rows_v, out_hbm.at[pl.ds(base, _CH)])
            return carry

        lax.fori_loop(0, _NCH, body, 0)

    return k(table, ids2d)


def _ln_body(g_ref, seg_ref, pos_ref, typ_ref, gam_ref, bet_ref, o_ref):
    x = g_ref[...] + pos_ref[...][None, :, :]
    seg = seg_ref[...]
    t = typ_ref[...]
    typ = jnp.where(seg == 1, t[1][None, None, :],
                    jnp.where(seg == 2, t[2][None, None, :],
                              t[0][None, None, :]))
    x = x + typ
    mean = jnp.mean(x, axis=-1, keepdims=True)
    var = jnp.mean(jnp.square(x - mean), axis=-1, keepdims=True)
    y = (x - mean) * lax.rsqrt(var + EPS)
    o_ref[...] = y * gam_ref[...][0][None, None, :] + bet_ref[...][0][None, None, :]


def _ln(gathered, segment_ids, pos_table, type_table, gamma, beta):
    BR = 16
    return pl.pallas_call(
        _ln_body,
        grid=(B // BR,),
        in_specs=[
            pl.BlockSpec((BR, MAX_LEN, DIM), lambda i: (i, 0, 0)),
            pl.BlockSpec((BR, MAX_LEN, 1), lambda i: (i, 0, 0)),
            pl.BlockSpec((MAX_LEN, DIM), lambda i: (0, 0)),
            pl.BlockSpec((3, DIM), lambda i: (0, 0)),
            pl.BlockSpec((1, DIM), lambda i: (0, 0)),
            pl.BlockSpec((1, DIM), lambda i: (0, 0)),
        ],
        out_specs=pl.BlockSpec((BR, MAX_LEN, DIM), lambda i: (i, 0, 0)),
        out_shape=jax.ShapeDtypeStruct((B, MAX_LEN, DIM), jnp.float32),
    )(gathered, segment_ids.reshape(B, MAX_LEN, 1), pos_table, type_table,
      gamma.reshape(1, DIM), beta.reshape(1, DIM))


def kernel(input_ids, segment_ids, word_table, W2, pos_table, type_table,
           gamma, beta):
    table = _transform_table(word_table, W2)
    ids2d = input_ids.astype(jnp.int32).reshape(TOK // 128, 128)
    gathered = _sc_gather(table, ids2d)
    return _ln(gathered.reshape(B, MAX_LEN, DIM), segment_ids.astype(jnp.int32),
               pos_table, type_table, gamma, beta)
